# baseline (speedup 1.0000x reference)
import functools

import jax
import jax.numpy as jnp
from jax import lax
from jax.experimental import pallas as pl
from jax.experimental.pallas import tpu as pltpu
from jax.experimental.pallas import tpu_sc as plsc

N = 10000
E = 320000
F = 128
G = 64
C = 16

NC = 2
NS = 16
EDGES_PER_TILE = E // (NC * NS)
EB = 80
NB = EDGES_PER_TILE // EB
CHUNK = 80
NCHUNK = N // CHUNK


def _lin1_body(x_ref, w_ref, b_ref, o_ref):
    o_ref[...] = lax.dot_general(
        x_ref[...], w_ref[...], (((1,), (1,)), ((), ())),
        preferred_element_type=jnp.float32) + b_ref[...]


def _lin2_body(p0_ref, p1_ref, w_ref, b_ref, o_ref):
    h = jnp.maximum(p0_ref[...] + p1_ref[...], 0.0)
    o_ref[...] = lax.dot_general(
        h, w_ref[...], (((1,), (1,)), ((), ())),
        preferred_element_type=jnp.float32) + b_ref[...]


def _head_body(q0_ref, q1_ref, batch_ref, wl_ref, bl_ref, o_ref):
    h = q0_ref[...] + q1_ref[...]
    b = batch_ref[...]
    oh = (b == lax.broadcasted_iota(jnp.int32, (N, G), 1)).astype(jnp.float32)
    sums = lax.dot_general(oh, h, (((0,), (0,)), ((), ())),
                           preferred_element_type=jnp.float32)
    ones = jnp.ones((N, 1), jnp.float32)
    counts = lax.dot_general(oh, ones, (((0,), (0,)), ((), ())),
                             preferred_element_type=jnp.float32)
    pooled = sums / jnp.maximum(counts, 1.0)
    logits = lax.dot_general(pooled, wl_ref[...], (((1,), (1,)), ((), ())),
                             preferred_element_type=jnp.float32) + bl_ref[...]
    m = jnp.max(logits, axis=1, keepdims=True)
    z = logits - m
    lse = jnp.log(jnp.sum(jnp.exp(z), axis=1, keepdims=True))
    o_ref[...] = z - lse


_R = 1000

_lin1 = pl.pallas_call(
    _lin1_body,
    grid=(N // _R,),
    in_specs=[
        pl.BlockSpec((_R, F), lambda i: (i, 0)),
        pl.BlockSpec((F, F), lambda i: (0, 0)),
        pl.BlockSpec((1, F), lambda i: (0, 0)),
    ],
    out_specs=pl.BlockSpec((_R, F), lambda i: (i, 0)),
    out_shape=jax.ShapeDtypeStruct((N, F), jnp.float32),
)

_lin2 = pl.pallas_call(
    _lin2_body,
    grid=(N // _R,),
    in_specs=[
        pl.BlockSpec((_R, F), lambda i: (i, 0)),
        pl.BlockSpec((_R, F), lambda i: (i, 0)),
        pl.BlockSpec((F, F), lambda i: (0, 0)),
        pl.BlockSpec((1, F), lambda i: (0, 0)),
    ],
    out_specs=pl.BlockSpec((_R, F), lambda i: (i, 0)),
    out_shape=jax.ShapeDtypeStruct((N, F), jnp.float32),
)

_head = pl.pallas_call(
    _head_body,
    out_shape=jax.ShapeDtypeStruct((G, C), jnp.float32),
)


HB1 = 63
HB2 = NB - HB1
HEMAX = HB1 * EB


def _prop_body(y_hbm, src_hbm, dst_hbm, w_hbm, out_hbm,
               src_t, dst_t, w_t, rows2, accum, gsem0, gsem1):
    c = lax.axis_index("c")
    s = lax.axis_index("s")
    wid = s * NC + c

    zeros16 = jnp.zeros((16,), jnp.float32)

    def zrow(r, carry):
        for k in range(F // 16):
            rows2[0, r, pl.ds(16 * k, 16)] = zeros16
        return carry

    lax.fori_loop(0, CHUNK, zrow, 0)
    for j in range((NCHUNK + NS - 1) // NS):
        idx = s + NS * j

        @pl.when(idx < NCHUNK)
        def _():
            off = pl.multiple_of(idx * CHUNK, 8)
            pltpu.sync_copy(rows2.at[0], accum.at[pl.ds(off, CHUNK)])

    plsc.subcore_barrier()

    ebase = pl.multiple_of(wid * EDGES_PER_TILE, 8)
    pltpu.sync_copy(dst_hbm.at[wid], dst_t)
    gsems = (gsem0, gsem1)

    def gather_issue(i_local, slot):
        pltpu.async_copy(
            y_hbm.at[src_t.at[pl.ds(i_local * EB, EB)]],
            rows2.at[slot], gsems[slot])

    def gather_wait(slot):
        pltpu.make_async_copy(
            y_hbm.at[src_t.at[pl.ds(0, EB)]],
            rows2.at[slot], gsems[slot]).wait()

    def by_parity(val, fn):
        @pl.when(val == 0)
        def _():
            fn(0)

        @pl.when(val == 1)
        def _():
            fn(1)

    for b0, nb in ((0, HB1), (HB1, HB2)):
        e0 = pl.multiple_of(ebase + b0 * EB, 8)
        ne = nb * EB
        pltpu.sync_copy(src_hbm.at[pl.ds(e0, ne)], src_t.at[pl.ds(0, ne)])
        pltpu.sync_copy(w_hbm.at[pl.ds(e0, ne)], w_t.at[pl.ds(0, ne)])
        gather_issue(0, 0)

        def body(i, carry):
            p = lax.rem(i, 2)

            @pl.when(i + 1 < nb)
            def _():
                def adv(slot):
                    gather_issue(i + 1, slot)
                by_parity(lax.rem(i + 1, 2), adv)

            by_parity(p, gather_wait)

            def scale(g, cc):
                wv16 = w_t[pl.ds(i * EB + g * 16, 16)]
                for j in range(16):
                    e = g * 16 + j
                    w = wv16[j]
                    for k in range(F // 16):
                        sl = pl.ds(16 * k, 16)
                        rows2[p, e, sl] = rows2[p, e, sl] * w
                return cc

            lax.fori_loop(0, EB // 16, scale, 0)
            pltpu.sync_copy(rows2.at[p], accum.at[dst_t.at[b0 + i]], add=True)
            return carry

        lax.fori_loop(0, nb, body, 0)

    plsc.subcore_barrier()

    for j in range((NCHUNK + NS - 1) // NS):
        idx = s + NS * j

        @pl.when(idx < NCHUNK)
        def _():
            off = pl.multiple_of(idx * CHUNK, 8)
            pltpu.sync_copy(accum.at[pl.ds(off, CHUNK)], rows2.at[0])
            pltpu.sync_copy(rows2.at[0], out_hbm.at[c, pl.ds(off, CHUNK)])


_propagate = functools.partial(
    pl.kernel,
    out_type=jax.ShapeDtypeStruct((NC, N, F), jnp.float32),
    mesh=plsc.VectorSubcoreMesh(core_axis_name="c", subcore_axis_name="s"),
    scratch_types=[
        pltpu.VMEM((HEMAX,), jnp.int32),
        pltpu.VMEM((NB, EB), jnp.int32),
        pltpu.VMEM((HEMAX,), jnp.float32),
        pltpu.VMEM((2, EB, F), jnp.float32),
        pltpu.VMEM_SHARED((N, F), jnp.float32),
        pltpu.SemaphoreType.DMA,
        pltpu.SemaphoreType.DMA,
    ],
)(_prop_body)


def kernel(x, edge_index, batch, w_mul, W1, b1, W2, b2, Wl, bl):
    src = edge_index[0]
    dst = edge_index[1].reshape(NC * NS, NB, EB)
    b1r = b1.reshape(1, F)
    b2r = b2.reshape(1, F)
    blr = bl.reshape(1, C)
    batch2 = batch.reshape(N, 1)

    y1 = _lin1(x, W1, b1r)
    p = _propagate(y1, src, dst, w_mul)
    y2 = _lin2(p[0], p[1], W2, b2r)
    q = _propagate(y2, src, dst, w_mul)
    return _head(q[0], q[1], batch2, Wl, blr)


# speedup vs baseline: 1.2191x; 1.2191x over previous
import functools

import jax
import jax.numpy as jnp
from jax import lax
from jax.experimental import pallas as pl
from jax.experimental.pallas import tpu as pltpu
from jax.experimental.pallas import tpu_sc as plsc

N = 10000
E = 320000
F = 128
G = 64
C = 16

NC = 2
NS = 16
EDGES_PER_TILE = E // (NC * NS)
EB = 128
NB = 79
EPT_P = NB * EB
HB1 = 40
HB2 = NB - HB1
HEMAX = HB1 * EB
CHUNK = 80
NCHUNK = N // CHUNK


def _lin1_body(x_ref, w_ref, b_ref, o_ref):
    o_ref[...] = lax.dot_general(
        x_ref[...], w_ref[...], (((1,), (1,)), ((), ())),
        preferred_element_type=jnp.float32) + b_ref[...]


def _lin2_body(p0_ref, p1_ref, w_ref, b_ref, o_ref):
    h = jnp.maximum(p0_ref[...] + p1_ref[...], 0.0)
    o_ref[...] = lax.dot_general(
        h, w_ref[...], (((1,), (1,)), ((), ())),
        preferred_element_type=jnp.float32) + b_ref[...]


def _head_body(q0_ref, q1_ref, batch_ref, wl_ref, bl_ref, o_ref):
    h = q0_ref[...] + q1_ref[...]
    b = batch_ref[...]
    oh = (b == lax.broadcasted_iota(jnp.int32, (N, G), 1)).astype(jnp.float32)
    sums = lax.dot_general(oh, h, (((0,), (0,)), ((), ())),
                           preferred_element_type=jnp.float32)
    ones = jnp.ones((N, 1), jnp.float32)
    counts = lax.dot_general(oh, ones, (((0,), (0,)), ((), ())),
                             preferred_element_type=jnp.float32)
    pooled = sums / jnp.maximum(counts, 1.0)
    logits = lax.dot_general(pooled, wl_ref[...], (((1,), (1,)), ((), ())),
                             preferred_element_type=jnp.float32) + bl_ref[...]
    m = jnp.max(logits, axis=1, keepdims=True)
    z = logits - m
    lse = jnp.log(jnp.sum(jnp.exp(z), axis=1, keepdims=True))
    o_ref[...] = z - lse


_R = 1000

_lin1 = pl.pallas_call(
    _lin1_body,
    grid=(N // _R,),
    in_specs=[
        pl.BlockSpec((_R, F), lambda i: (i, 0)),
        pl.BlockSpec((F, F), lambda i: (0, 0)),
        pl.BlockSpec((1, F), lambda i: (0, 0)),
    ],
    out_specs=pl.BlockSpec((_R, F), lambda i: (i, 0)),
    out_shape=jax.ShapeDtypeStruct((N, F), jnp.float32),
)

_lin2 = pl.pallas_call(
    _lin2_body,
    grid=(N // _R,),
    in_specs=[
        pl.BlockSpec((_R, F), lambda i: (i, 0)),
        pl.BlockSpec((_R, F), lambda i: (i, 0)),
        pl.BlockSpec((F, F), lambda i: (0, 0)),
        pl.BlockSpec((1, F), lambda i: (0, 0)),
    ],
    out_specs=pl.BlockSpec((_R, F), lambda i: (i, 0)),
    out_shape=jax.ShapeDtypeStruct((N, F), jnp.float32),
)

_head = pl.pallas_call(
    _head_body,
    out_shape=jax.ShapeDtypeStruct((G, C), jnp.float32),
)


def _prop_body(y_hbm, src_hbm, dst_hbm, w_hbm, out_hbm,
               src_t, dst_t, w_t, rows, accum, sem):
    c = lax.axis_index("c")
    s = lax.axis_index("s")
    wid = s * NC + c

    zeros16 = jnp.zeros((16,), jnp.float32)

    def zrow(r, carry):
        for k in range(F // 16):
            rows[r, pl.ds(16 * k, 16)] = zeros16
        return carry

    lax.fori_loop(0, CHUNK, zrow, 0)
    for j in range((NCHUNK + NS - 1) // NS):
        idx = s + NS * j

        @pl.when(idx < NCHUNK)
        def _():
            off = pl.multiple_of(idx * CHUNK, 8)
            pltpu.sync_copy(rows.at[pl.ds(0, CHUNK)], accum.at[pl.ds(off, CHUNK)])

    plsc.subcore_barrier()

    ebase = pl.multiple_of(wid * EPT_P, 8)
    pltpu.sync_copy(dst_hbm.at[wid], dst_t)

    for b0, nb in ((0, HB1), (HB1, HB2)):
        e0 = pl.multiple_of(ebase + b0 * EB, 8)
        ne = nb * EB
        pltpu.sync_copy(src_hbm.at[pl.ds(e0, ne)], src_t.at[pl.ds(0, ne)])
        pltpu.sync_copy(w_hbm.at[pl.ds(e0, ne)], w_t.at[pl.ds(0, ne)])

        def body(i, carry):
            pltpu.async_copy(
                y_hbm.at[src_t.at[pl.ds(i * EB, EB)]], rows, sem).wait()

            def scale(g, cc):
                wv16 = w_t[pl.ds(i * EB + g * 16, 16)]
                for j in range(16):
                    e = g * 16 + j
                    w = wv16[j]
                    for k in range(F // 16):
                        sl = pl.ds(16 * k, 16)
                        rows[e, sl] = rows[e, sl] * w
                return cc

            lax.fori_loop(0, EB // 16, scale, 0)
            pltpu.sync_copy(rows, accum.at[dst_t.at[b0 + i]], add=True)
            return carry

        lax.fori_loop(0, nb, body, 0)

    plsc.subcore_barrier()

    for j in range((NCHUNK + NS - 1) // NS):
        idx = s + NS * j

        @pl.when(idx < NCHUNK)
        def _():
            off = pl.multiple_of(idx * CHUNK, 8)
            pltpu.sync_copy(accum.at[pl.ds(off, CHUNK)], rows.at[pl.ds(0, CHUNK)])
            pltpu.sync_copy(rows.at[pl.ds(0, CHUNK)], out_hbm.at[c, pl.ds(off, CHUNK)])


_propagate = functools.partial(
    pl.kernel,
    out_type=jax.ShapeDtypeStruct((NC, N, F), jnp.float32),
    mesh=plsc.VectorSubcoreMesh(core_axis_name="c", subcore_axis_name="s"),
    scratch_types=[
        pltpu.VMEM((HEMAX,), jnp.int32),
        pltpu.VMEM((NB, EB), jnp.int32),
        pltpu.VMEM((HEMAX,), jnp.float32),
        pltpu.VMEM((EB, F), jnp.float32),
        pltpu.VMEM_SHARED((N, F), jnp.float32),
        pltpu.SemaphoreType.DMA,
    ],
)(_prop_body)


def kernel(x, edge_index, batch, w_mul, W1, b1, W2, b2, Wl, bl):
    padcfg = ((0, 0), (0, EPT_P - EDGES_PER_TILE))
    src = jnp.pad(edge_index[0].reshape(NC * NS, EDGES_PER_TILE),
                  padcfg).reshape(-1)
    dst = jnp.pad(edge_index[1].reshape(NC * NS, EDGES_PER_TILE),
                  padcfg).reshape(NC * NS, NB, EB)
    wp = jnp.pad(w_mul.reshape(NC * NS, EDGES_PER_TILE), padcfg).reshape(-1)
    b1r = b1.reshape(1, F)
    b2r = b2.reshape(1, F)
    blr = bl.reshape(1, C)
    batch2 = batch.reshape(N, 1)

    y1 = _lin1(x, W1, b1r)
    p = _propagate(y1, src, dst, wp)
    y2 = _lin2(p[0], p[1], W2, b2r)
    q = _propagate(y2, src, dst, wp)
    return _head(q[0], q[1], batch2, Wl, blr)


# speedup vs baseline: 1.6309x; 1.3378x over previous
import functools

import jax
import jax.numpy as jnp
from jax import lax
from jax.experimental import pallas as pl
from jax.experimental.pallas import tpu as pltpu
from jax.experimental.pallas import tpu_sc as plsc

N = 10000
E = 320000
F = 128
G = 64
C = 16

NC = 2
NS = 16
EDGES_PER_TILE = E // (NC * NS)
EB = 80
NB = EDGES_PER_TILE // EB
CHUNK = 80
NCHUNK = N // CHUNK


def _lin1_body(x_ref, w_ref, b_ref, o_ref):
    o_ref[...] = lax.dot_general(
        x_ref[...], w_ref[...], (((1,), (1,)), ((), ())),
        preferred_element_type=jnp.float32) + b_ref[...]


def _lin2_body(p0_ref, p1_ref, w_ref, b_ref, o_ref):
    h = jnp.maximum(p0_ref[...] + p1_ref[...], 0.0)
    o_ref[...] = lax.dot_general(
        h, w_ref[...], (((1,), (1,)), ((), ())),
        preferred_element_type=jnp.float32) + b_ref[...]


def _head_body(q0_ref, q1_ref, batch_ref, wl_ref, bl_ref, o_ref):
    h = q0_ref[...] + q1_ref[...]
    b = batch_ref[...]
    oh = (b == lax.broadcasted_iota(jnp.int32, (N, G), 1)).astype(jnp.float32)
    sums = lax.dot_general(oh, h, (((0,), (0,)), ((), ())),
                           preferred_element_type=jnp.float32)
    ones = jnp.ones((N, 1), jnp.float32)
    counts = lax.dot_general(oh, ones, (((0,), (0,)), ((), ())),
                             preferred_element_type=jnp.float32)
    pooled = sums / jnp.maximum(counts, 1.0)
    logits = lax.dot_general(pooled, wl_ref[...], (((1,), (1,)), ((), ())),
                             preferred_element_type=jnp.float32) + bl_ref[...]
    m = jnp.max(logits, axis=1, keepdims=True)
    z = logits - m
    lse = jnp.log(jnp.sum(jnp.exp(z), axis=1, keepdims=True))
    o_ref[...] = z - lse


_R = 1000

_lin1 = pl.pallas_call(
    _lin1_body,
    grid=(N // _R,),
    in_specs=[
        pl.BlockSpec((_R, F), lambda i: (i, 0)),
        pl.BlockSpec((F, F), lambda i: (0, 0)),
        pl.BlockSpec((1, F), lambda i: (0, 0)),
    ],
    out_specs=pl.BlockSpec((_R, F), lambda i: (i, 0)),
    out_shape=jax.ShapeDtypeStruct((N, F), jnp.float32),
)

_lin2 = pl.pallas_call(
    _lin2_body,
    grid=(N // _R,),
    in_specs=[
        pl.BlockSpec((_R, F), lambda i: (i, 0)),
        pl.BlockSpec((_R, F), lambda i: (i, 0)),
        pl.BlockSpec((F, F), lambda i: (0, 0)),
        pl.BlockSpec((1, F), lambda i: (0, 0)),
    ],
    out_specs=pl.BlockSpec((_R, F), lambda i: (i, 0)),
    out_shape=jax.ShapeDtypeStruct((N, F), jnp.float32),
)

_head = pl.pallas_call(
    _head_body,
    out_shape=jax.ShapeDtypeStruct((G, C), jnp.float32),
)


def _prop_body(y_hbm, src_hbm, dst_hbm, w_hbm, out_hbm,
               src_t, dst_t, w_t, rows, accum, sem):
    c = lax.axis_index("c")
    s = lax.axis_index("s")
    wid = s * NC + c

    zeros16 = jnp.zeros((16,), jnp.float32)

    @plsc.parallel_loop(0, CHUNK, 1, unroll=4)
    def _(r):
        for k in range(F // 16):
            rows[r, pl.ds(16 * k, 16)] = zeros16
    for j in range((NCHUNK + NS - 1) // NS):
        idx = s + NS * j

        @pl.when(idx < NCHUNK)
        def _():
            off = pl.multiple_of(idx * CHUNK, 8)
            pltpu.sync_copy(rows, accum.at[pl.ds(off, CHUNK)])

    plsc.subcore_barrier()

    ebase = pl.multiple_of(wid * EDGES_PER_TILE, 8)
    pltpu.sync_copy(src_hbm.at[pl.ds(ebase, EDGES_PER_TILE)], src_t)
    pltpu.sync_copy(w_hbm.at[pl.ds(ebase, EDGES_PER_TILE)], w_t)
    pltpu.sync_copy(dst_hbm.at[wid], dst_t)

    def body(i, carry):
        pltpu.async_copy(
            y_hbm.at[src_t.at[pl.ds(i * EB, EB)]], rows, sem).wait()

        @plsc.parallel_loop(0, EB // 16, 1, unroll=2)
        def _(g):
            wv16 = w_t[pl.ds(i * EB + g * 16, 16)]
            for j in range(16):
                e = g * 16 + j
                w = wv16[j]
                for k in range(F // 16):
                    sl = pl.ds(16 * k, 16)
                    rows[e, sl] = rows[e, sl] * w
        pltpu.sync_copy(rows, accum.at[dst_t.at[i]], add=True)
        return carry

    lax.fori_loop(0, NB, body, 0)
    plsc.subcore_barrier()

    for j in range((NCHUNK + NS - 1) // NS):
        idx = s + NS * j

        @pl.when(idx < NCHUNK)
        def _():
            off = pl.multiple_of(idx * CHUNK, 8)
            pltpu.sync_copy(accum.at[pl.ds(off, CHUNK)], rows)
            pltpu.sync_copy(rows, out_hbm.at[c, pl.ds(off, CHUNK)])


_propagate = functools.partial(
    pl.kernel,
    out_type=jax.ShapeDtypeStruct((NC, N, F), jnp.float32),
    mesh=plsc.VectorSubcoreMesh(core_axis_name="c", subcore_axis_name="s"),
    scratch_types=[
        pltpu.VMEM((EDGES_PER_TILE,), jnp.int32),
        pltpu.VMEM((NB, EB), jnp.int32),
        pltpu.VMEM((EDGES_PER_TILE,), jnp.float32),
        pltpu.VMEM((EB, F), jnp.float32),
        pltpu.VMEM_SHARED((N, F), jnp.float32),
        pltpu.SemaphoreType.DMA,
    ],
)(_prop_body)


def kernel(x, edge_index, batch, w_mul, W1, b1, W2, b2, Wl, bl):
    src = edge_index[0]
    dst = edge_index[1].reshape(NC * NS, NB, EB)
    b1r = b1.reshape(1, F)
    b2r = b2.reshape(1, F)
    blr = bl.reshape(1, C)
    batch2 = batch.reshape(N, 1)

    y1 = _lin1(x, W1, b1r)
    p = _propagate(y1, src, dst, w_mul)
    y2 = _lin2(p[0], p[1], W2, b2r)
    q = _propagate(y2, src, dst, w_mul)
    return _head(q[0], q[1], batch2, Wl, blr)


# speedup vs baseline: 1.9216x; 1.1783x over previous
import functools

import jax
import jax.numpy as jnp
from jax import lax
from jax.experimental import pallas as pl
from jax.experimental.pallas import tpu as pltpu
from jax.experimental.pallas import tpu_sc as plsc

N = 10000
E = 320000
F = 128
G = 64
C = 16

NC = 2
NS = 16
EDGES_PER_TILE = E // (NC * NS)
EB = 80
NB = EDGES_PER_TILE // EB
HB1 = 63
HB2 = NB - HB1
HEMAX = HB1 * EB
CHUNK = 80
NCHUNK = N // CHUNK


def _lin1_body(x_ref, w_ref, b_ref, o_ref):
    o_ref[...] = lax.dot_general(
        x_ref[...], w_ref[...], (((1,), (1,)), ((), ())),
        preferred_element_type=jnp.float32) + b_ref[...]


def _lin2_body(p0_ref, p1_ref, w_ref, b_ref, o_ref):
    h = jnp.maximum(p0_ref[...] + p1_ref[...], 0.0)
    o_ref[...] = lax.dot_general(
        h, w_ref[...], (((1,), (1,)), ((), ())),
        preferred_element_type=jnp.float32) + b_ref[...]


def _head_body(q0_ref, q1_ref, batch_ref, wl_ref, bl_ref, o_ref):
    h = q0_ref[...] + q1_ref[...]
    b = batch_ref[...]
    oh = (b == lax.broadcasted_iota(jnp.int32, (N, G), 1)).astype(jnp.float32)
    sums = lax.dot_general(oh, h, (((0,), (0,)), ((), ())),
                           preferred_element_type=jnp.float32)
    ones = jnp.ones((N, 1), jnp.float32)
    counts = lax.dot_general(oh, ones, (((0,), (0,)), ((), ())),
                             preferred_element_type=jnp.float32)
    pooled = sums / jnp.maximum(counts, 1.0)
    logits = lax.dot_general(pooled, wl_ref[...], (((1,), (1,)), ((), ())),
                             preferred_element_type=jnp.float32) + bl_ref[...]
    m = jnp.max(logits, axis=1, keepdims=True)
    z = logits - m
    lse = jnp.log(jnp.sum(jnp.exp(z), axis=1, keepdims=True))
    o_ref[...] = z - lse


_R = 1000

_lin1 = pl.pallas_call(
    _lin1_body,
    grid=(N // _R,),
    in_specs=[
        pl.BlockSpec((_R, F), lambda i: (i, 0)),
        pl.BlockSpec((F, F), lambda i: (0, 0)),
        pl.BlockSpec((1, F), lambda i: (0, 0)),
    ],
    out_specs=pl.BlockSpec((_R, F), lambda i: (i, 0)),
    out_shape=jax.ShapeDtypeStruct((N, F), jnp.float32),
)

_lin2 = pl.pallas_call(
    _lin2_body,
    grid=(N // _R,),
    in_specs=[
        pl.BlockSpec((_R, F), lambda i: (i, 0)),
        pl.BlockSpec((_R, F), lambda i: (i, 0)),
        pl.BlockSpec((F, F), lambda i: (0, 0)),
        pl.BlockSpec((1, F), lambda i: (0, 0)),
    ],
    out_specs=pl.BlockSpec((_R, F), lambda i: (i, 0)),
    out_shape=jax.ShapeDtypeStruct((N, F), jnp.float32),
)

_head = pl.pallas_call(
    _head_body,
    out_shape=jax.ShapeDtypeStruct((G, C), jnp.float32),
)


def _prop_body(y_hbm, src_hbm, dst_hbm, w_hbm, out_hbm,
               src_t, dst_t, w_t, rows2, accum, sem0, sem1):
    c = lax.axis_index("c")
    s = lax.axis_index("s")
    wid = s * NC + c

    zeros16 = jnp.zeros((16,), jnp.float32)

    @plsc.parallel_loop(0, CHUNK, 1, unroll=4)
    def _(r):
        for k in range(F // 16):
            rows2[0, r, pl.ds(16 * k, 16)] = zeros16
    for j in range((NCHUNK + NS - 1) // NS):
        idx = s + NS * j

        @pl.when(idx < NCHUNK)
        def _():
            off = pl.multiple_of(idx * CHUNK, 8)
            pltpu.sync_copy(rows2.at[0], accum.at[pl.ds(off, CHUNK)])

    plsc.subcore_barrier()

    ebase = pl.multiple_of(wid * EDGES_PER_TILE, 8)
    pltpu.sync_copy(dst_hbm.at[wid], dst_t)

    def proc(i_local, i_global, buf):
        @plsc.parallel_loop(0, EB // 16, 1, unroll=2)
        def _(g):
            wv16 = w_t[pl.ds(i_local * EB + g * 16, 16)]
            for j in range(16):
                e = g * 16 + j
                w = wv16[j]
                for k in range(F // 16):
                    sl = pl.ds(16 * k, 16)
                    rows2[buf, e, sl] = rows2[buf, e, sl] * w

        pltpu.sync_copy(rows2.at[buf], accum.at[dst_t.at[i_global]], add=True)

    for b0, nb in ((0, HB1), (HB1, HB2)):
        e0 = pl.multiple_of(ebase + b0 * EB, 8)
        ne = nb * EB
        pltpu.sync_copy(src_hbm.at[pl.ds(e0, ne)], src_t.at[pl.ds(0, ne)])
        pltpu.sync_copy(w_hbm.at[pl.ds(e0, ne)], w_t.at[pl.ds(0, ne)])

        def pair(t, carry):
            i0 = 2 * t
            i1 = 2 * t + 1
            d0 = pltpu.async_copy(
                y_hbm.at[src_t.at[pl.ds(i0 * EB, EB)]], rows2.at[0], sem0)
            d1 = pltpu.async_copy(
                y_hbm.at[src_t.at[pl.ds(i1 * EB, EB)]], rows2.at[1], sem1)
            d0.wait()
            proc(i0, b0 + i0, 0)
            d1.wait()
            proc(i1, b0 + i1, 1)
            return carry

        lax.fori_loop(0, nb // 2, pair, 0)
        if nb % 2:
            il = nb - 1
            pltpu.async_copy(
                y_hbm.at[src_t.at[pl.ds(il * EB, EB)]],
                rows2.at[0], sem0).wait()
            proc(il, b0 + il, 0)

    plsc.subcore_barrier()

    for j in range((NCHUNK + NS - 1) // NS):
        idx = s + NS * j

        @pl.when(idx < NCHUNK)
        def _():
            off = pl.multiple_of(idx * CHUNK, 8)
            pltpu.sync_copy(accum.at[pl.ds(off, CHUNK)], rows2.at[0])
            pltpu.sync_copy(rows2.at[0], out_hbm.at[c, pl.ds(off, CHUNK)])


_propagate = functools.partial(
    pl.kernel,
    out_type=jax.ShapeDtypeStruct((NC, N, F), jnp.float32),
    mesh=plsc.VectorSubcoreMesh(core_axis_name="c", subcore_axis_name="s"),
    scratch_types=[
        pltpu.VMEM((HEMAX,), jnp.int32),
        pltpu.VMEM((NB, EB), jnp.int32),
        pltpu.VMEM((HEMAX,), jnp.float32),
        pltpu.VMEM((2, EB, F), jnp.float32),
        pltpu.VMEM_SHARED((N, F), jnp.float32),
        pltpu.SemaphoreType.DMA,
        pltpu.SemaphoreType.DMA,
    ],
)(_prop_body)


def kernel(x, edge_index, batch, w_mul, W1, b1, W2, b2, Wl, bl):
    src = edge_index[0]
    dst = edge_index[1].reshape(NC * NS, NB, EB)
    b1r = b1.reshape(1, F)
    b2r = b2.reshape(1, F)
    blr = bl.reshape(1, C)
    batch2 = batch.reshape(N, 1)

    y1 = _lin1(x, W1, b1r)
    p = _propagate(y1, src, dst, w_mul)
    y2 = _lin2(p[0], p[1], W2, b2r)
    q = _propagate(y2, src, dst, w_mul)
    return _head(q[0], q[1], batch2, Wl, blr)


# speedup vs baseline: 2.6039x; 1.3550x over previous
import functools

import jax
import jax.numpy as jnp
from jax import lax
from jax.experimental import pallas as pl
from jax.experimental.pallas import tpu as pltpu
from jax.experimental.pallas import tpu_sc as plsc

N = 10000
E = 320000
F = 128
G = 64
C = 16

NC = 2
NS = 16
EDGES_PER_TILE = E // (NC * NS)
EB = 80
NB = EDGES_PER_TILE // EB
HB1 = 63
HB2 = NB - HB1
HEMAX = HB1 * EB
CHUNK = 80
NCHUNK = N // CHUNK


def _lin1_body(x_ref, w_ref, b_ref, o_ref):
    o_ref[...] = lax.dot_general(
        x_ref[...], w_ref[...], (((1,), (1,)), ((), ())),
        preferred_element_type=jnp.float32) + b_ref[...]


def _lin2_body(p0_ref, p1_ref, w_ref, b_ref, o_ref):
    h = jnp.maximum(p0_ref[...] + p1_ref[...], 0.0)
    o_ref[...] = lax.dot_general(
        h, w_ref[...], (((1,), (1,)), ((), ())),
        preferred_element_type=jnp.float32) + b_ref[...]


def _head_body(q0_ref, q1_ref, batch_ref, wl_ref, bl_ref, o_ref):
    h = q0_ref[...] + q1_ref[...]
    b = batch_ref[...]
    oh = (b == lax.broadcasted_iota(jnp.int32, (N, G), 1)).astype(jnp.float32)
    sums = lax.dot_general(oh, h, (((0,), (0,)), ((), ())),
                           preferred_element_type=jnp.float32)
    ones = jnp.ones((N, 1), jnp.float32)
    counts = lax.dot_general(oh, ones, (((0,), (0,)), ((), ())),
                             preferred_element_type=jnp.float32)
    pooled = sums / jnp.maximum(counts, 1.0)
    logits = lax.dot_general(pooled, wl_ref[...], (((1,), (1,)), ((), ())),
                             preferred_element_type=jnp.float32) + bl_ref[...]
    m = jnp.max(logits, axis=1, keepdims=True)
    z = logits - m
    lse = jnp.log(jnp.sum(jnp.exp(z), axis=1, keepdims=True))
    o_ref[...] = z - lse


_R = 1000

_lin1 = pl.pallas_call(
    _lin1_body,
    grid=(N // _R,),
    in_specs=[
        pl.BlockSpec((_R, F), lambda i: (i, 0)),
        pl.BlockSpec((F, F), lambda i: (0, 0)),
        pl.BlockSpec((1, F), lambda i: (0, 0)),
    ],
    out_specs=pl.BlockSpec((_R, F), lambda i: (i, 0)),
    out_shape=jax.ShapeDtypeStruct((N, F), jnp.float32),
)

_lin2 = pl.pallas_call(
    _lin2_body,
    grid=(N // _R,),
    in_specs=[
        pl.BlockSpec((_R, F), lambda i: (i, 0)),
        pl.BlockSpec((_R, F), lambda i: (i, 0)),
        pl.BlockSpec((F, F), lambda i: (0, 0)),
        pl.BlockSpec((1, F), lambda i: (0, 0)),
    ],
    out_specs=pl.BlockSpec((_R, F), lambda i: (i, 0)),
    out_shape=jax.ShapeDtypeStruct((N, F), jnp.float32),
)

_head = pl.pallas_call(
    _head_body,
    out_shape=jax.ShapeDtypeStruct((G, C), jnp.float32),
)


def _prop_body(y_hbm, src_hbm, dst_hbm, w_hbm, out_hbm,
               src_t, dst_t, w_t, rows2, accum, sem0, sem1):
    c = lax.axis_index("c")
    s = lax.axis_index("s")
    wid = s * NC + c

    zeros16 = jnp.zeros((16,), jnp.float32)

    @plsc.parallel_loop(0, CHUNK, 1, unroll=4)
    def _(r):
        for k in range(F // 16):
            rows2[0, r, pl.ds(16 * k, 16)] = zeros16
    for j in range((NCHUNK + NS - 1) // NS):
        idx = s + NS * j

        @pl.when(idx < NCHUNK)
        def _():
            off = pl.multiple_of(idx * CHUNK, 8)
            pltpu.sync_copy(rows2.at[0], accum.at[pl.ds(off, CHUNK)])

    plsc.subcore_barrier()

    ebase = pl.multiple_of(wid * EDGES_PER_TILE, 8)
    pltpu.sync_copy(dst_hbm.at[wid], dst_t)

    def proc(i_local, i_global, buf):
        @plsc.parallel_loop(0, EB // 16, 1, unroll=2)
        def _(g):
            wv16 = w_t[pl.ds(i_local * EB + g * 16, 16)]
            for j in range(16):
                e = g * 16 + j
                w = wv16[j]
                for k in range(F // 16):
                    sl = pl.ds(16 * k, 16)
                    rows2[buf, e, sl] = rows2[buf, e, sl] * w

        pltpu.sync_copy(rows2.at[buf], accum.at[dst_t.at[i_global]], add=True)

    for b0, nb in ((0, HB1), (HB1, HB2)):
        e0 = pl.multiple_of(ebase + b0 * EB, 8)
        ne = nb * EB
        pltpu.sync_copy(src_hbm.at[pl.ds(e0, ne)], src_t.at[pl.ds(0, ne)])
        pltpu.sync_copy(w_hbm.at[pl.ds(e0, ne)], w_t.at[pl.ds(0, ne)])

        pltpu.async_copy(
            y_hbm.at[src_t.at[pl.ds(0, EB)]], rows2.at[0], sem0)

        def pair(t, carry):
            i0 = 2 * t
            i1 = 2 * t + 1
            d1 = pltpu.async_copy(
                y_hbm.at[src_t.at[pl.ds(i1 * EB, EB)]], rows2.at[1], sem1)
            pltpu.make_async_copy(
                y_hbm.at[src_t.at[pl.ds(0, EB)]], rows2.at[0], sem0).wait()
            proc(i0, b0 + i0, 0)

            @pl.when(i0 + 2 < nb)
            def _():
                pltpu.async_copy(
                    y_hbm.at[src_t.at[pl.ds((i0 + 2) * EB, EB)]],
                    rows2.at[0], sem0)

            d1.wait()
            proc(i1, b0 + i1, 1)
            return carry

        lax.fori_loop(0, nb // 2, pair, 0)
        if nb % 2:
            il = nb - 1
            pltpu.make_async_copy(
                y_hbm.at[src_t.at[pl.ds(0, EB)]], rows2.at[0], sem0).wait()
            proc(il, b0 + il, 0)

    plsc.subcore_barrier()

    for j in range((NCHUNK + NS - 1) // NS):
        idx = s + NS * j

        @pl.when(idx < NCHUNK)
        def _():
            off = pl.multiple_of(idx * CHUNK, 8)
            pltpu.sync_copy(accum.at[pl.ds(off, CHUNK)], rows2.at[0])
            pltpu.sync_copy(rows2.at[0], out_hbm.at[c, pl.ds(off, CHUNK)])


_propagate = functools.partial(
    pl.kernel,
    out_type=jax.ShapeDtypeStruct((NC, N, F), jnp.float32),
    mesh=plsc.VectorSubcoreMesh(core_axis_name="c", subcore_axis_name="s"),
    scratch_types=[
        pltpu.VMEM((HEMAX,), jnp.int32),
        pltpu.VMEM((NB, EB), jnp.int32),
        pltpu.VMEM((HEMAX,), jnp.float32),
        pltpu.VMEM((2, EB, F), jnp.float32),
        pltpu.VMEM_SHARED((N, F), jnp.float32),
        pltpu.SemaphoreType.DMA,
        pltpu.SemaphoreType.DMA,
    ],
)(_prop_body)


def kernel(x, edge_index, batch, w_mul, W1, b1, W2, b2, Wl, bl):
    src = edge_index[0]
    dst = edge_index[1].reshape(NC * NS, NB, EB)
    b1r = b1.reshape(1, F)
    b2r = b2.reshape(1, F)
    blr = bl.reshape(1, C)
    batch2 = batch.reshape(N, 1)

    y1 = _lin1(x, W1, b1r)
    p = _propagate(y1, src, dst, w_mul)
    y2 = _lin2(p[0], p[1], W2, b2r)
    q = _propagate(y2, src, dst, w_mul)
    return _head(q[0], q[1], batch2, Wl, blr)
